# Initial kernel scaffold; baseline (speedup 1.0000x reference)
import jax
import jax.numpy as jnp
from jax.experimental import pallas as pl


def kernel(x, edge_index, edge_attr, pkt_mask, batch, W_e, b_e, eps, W1, b1, W2, b2, gamma, beta, W_v, b_v):
    raise NotImplementedError("write your pallas kernel here")


# speedup vs baseline: 2.5986x; 2.5986x over previous
import functools

import jax
import jax.numpy as jnp
from jax import lax
from jax.experimental import pallas as pl
from jax.experimental.pallas import tpu as pltpu
from jax.experimental.pallas import tpu_sc as plsc

N = 10000
E = 320000
D = 128
DE = 16
K = 8
B = 16

NC = 2
NS = 16
L = 16
NW = NC * NS
EPW = E // NW
C = 80
NCH = EPW // C
NP = 10240
RPS = NP // NS

R = 1000
NB = N // R


def _emb_body(a_ref, w_ref, b_ref, o_ref):
    o_ref[...] = (
        jnp.dot(a_ref[...], w_ref[...], preferred_element_type=jnp.float32)
        + b_ref[...]
    )


def _edge_emb(edge_attr, W_e, b_e2):
    BE = 8000
    return pl.pallas_call(
        _emb_body,
        grid=(E // BE,),
        in_specs=[
            pl.BlockSpec((BE, DE), lambda i: (i, 0)),
            pl.BlockSpec((DE, D), lambda i: (0, 0)),
            pl.BlockSpec((1, D), lambda i: (0, 0)),
        ],
        out_specs=pl.BlockSpec((BE, D), lambda i: (i, 0)),
        out_shape=jax.ShapeDtypeStruct((E, D), jnp.float32),
    )(edge_attr, W_e, b_e2)


def _sc_body(x_hbm, src_hbm, dst_hbm, emb_hbm, zero_hbm, out_hbm,
             src_v, dst_v, rows_v, emb_v, agg_sh, sem):
    cid = lax.axis_index("c")
    sid = lax.axis_index("s")
    wid = cid * NS + sid
    row0 = sid * RPS

    pltpu.sync_copy(zero_hbm.at[pl.ds(row0, RPS)],
                    agg_sh.at[pl.ds(row0, RPS)])
    plsc.subcore_barrier()

    e0 = wid * EPW

    def chunk(t, carry):
        off = e0 + t * C
        pltpu.sync_copy(src_hbm.at[pl.ds(off, C)], src_v)
        pltpu.sync_copy(dst_hbm.at[pl.ds(off, C)], dst_v)
        pltpu.sync_copy(emb_hbm.at[pl.ds(off, C)], emb_v)
        pltpu.async_copy(x_hbm.at[src_v], rows_v, sem).wait()

        def row(i, c2):
            for jj in range(D // L):
                sl = pl.ds(jj * L, L)
                rows_v[i, sl] = jnp.maximum(rows_v[i, sl] + emb_v[i, sl], 0.0)
            return c2

        lax.fori_loop(0, C, row, 0)
        pltpu.sync_copy(rows_v, agg_sh.at[dst_v], add=True)
        return carry

    lax.fori_loop(0, NCH, chunk, 0)
    plsc.subcore_barrier()
    pltpu.sync_copy(agg_sh.at[pl.ds(row0, RPS)],
                    out_hbm.at[pl.ds(cid * NP + row0, RPS)])


_sc_agg = functools.partial(
    pl.kernel,
    out_type=jax.ShapeDtypeStruct((NC * NP, D), jnp.float32),
    mesh=plsc.VectorSubcoreMesh(
        core_axis_name="c", subcore_axis_name="s",
        num_cores=NC, num_subcores=NS),
    scratch_types=[
        pltpu.VMEM((C,), jnp.int32),
        pltpu.VMEM((C,), jnp.int32),
        pltpu.VMEM((C, D), jnp.float32),
        pltpu.VMEM((C, D), jnp.float32),
        pltpu.VMEM_SHARED((NP, D), jnp.float32),
        pltpu.SemaphoreType.DMA,
    ],
)(_sc_body)


def _mask_matrix(bf, pm):
    c = lax.broadcasted_iota(jnp.int32, (R, B * K), 1)
    onehot = bf == (c // K).astype(jnp.float32)
    pmt = jnp.concatenate([pm] * B, axis=1)
    return jnp.where(onehot, pmt, 0.0)


def _s3a_body(x_ref, a0_ref, a1_ref, bf_ref, pm_ref, eps_ref,
              w1_ref, b1_ref, w2_ref, b2_ref,
              xr_ref, p2t_ref, cnt_ref):
    i = pl.program_id(0)
    h = x_ref[...] * (1.0 + eps_ref[0, 0]) + a0_ref[...] + a1_ref[...]
    t = jnp.maximum(
        jnp.dot(h, w1_ref[...], preferred_element_type=jnp.float32)
        + b1_ref[...], 0.0)
    xr = (jnp.dot(t, w2_ref[...], preferred_element_type=jnp.float32)
          + b2_ref[...])
    xr_ref[...] = xr

    M = _mask_matrix(bf_ref[...], pm_ref[...])

    @pl.when(i == 0)
    def _():
        p2t_ref[...] = jnp.zeros_like(p2t_ref)
        cnt_ref[...] = jnp.zeros_like(cnt_ref)

    p2t_ref[...] += lax.dot_general(
        xr, M, (((0,), (0,)), ((), ())), preferred_element_type=jnp.float32)
    cnt_ref[...] += jnp.sum(M, axis=0, keepdims=True)


def _s3b_body(xr_ref, bf_ref, pm_ref, p2t_ref, cnt_ref,
              wv_ref, bv_ref, g_ref, be_ref, o_ref):
    recip = 1.0 / (cnt_ref[...] + 1e-9)
    pmean_t = p2t_ref[...] * recip
    pe = lax.dot_general(
        pmean_t, wv_ref[...], (((0,), (0,)), ((), ())),
        preferred_element_type=jnp.float32) + bv_ref[...]
    M = _mask_matrix(bf_ref[...], pm_ref[...])
    v = xr_ref[...] + jnp.dot(M, pe, preferred_element_type=jnp.float32)
    mu = jnp.mean(v, axis=1, keepdims=True)
    d = v - mu
    var = jnp.mean(d * d, axis=1, keepdims=True)
    o = d * lax.rsqrt(var + 1e-5) * g_ref[...] + be_ref[...]
    o_ref[...] = jnp.maximum(o, 0.0)


def _stage3a(x, a0, a1, bf, pm, eps2, W1, b12, W2, b22):
    return pl.pallas_call(
        _s3a_body,
        grid=(NB,),
        in_specs=[
            pl.BlockSpec((R, D), lambda i: (i, 0)),
            pl.BlockSpec((R, D), lambda i: (i, 0)),
            pl.BlockSpec((R, D), lambda i: (i, 0)),
            pl.BlockSpec((R, 1), lambda i: (i, 0)),
            pl.BlockSpec((R, K), lambda i: (i, 0)),
            pl.BlockSpec((1, 1), lambda i: (0, 0)),
            pl.BlockSpec((D, D), lambda i: (0, 0)),
            pl.BlockSpec((1, D), lambda i: (0, 0)),
            pl.BlockSpec((D, D), lambda i: (0, 0)),
            pl.BlockSpec((1, D), lambda i: (0, 0)),
        ],
        out_specs=[
            pl.BlockSpec((R, D), lambda i: (i, 0)),
            pl.BlockSpec((D, B * K), lambda i: (0, 0)),
            pl.BlockSpec((1, B * K), lambda i: (0, 0)),
        ],
        out_shape=[
            jax.ShapeDtypeStruct((N, D), jnp.float32),
            jax.ShapeDtypeStruct((D, B * K), jnp.float32),
            jax.ShapeDtypeStruct((1, B * K), jnp.float32),
        ],
    )(x, a0, a1, bf, pm, eps2, W1, b12, W2, b22)


def _stage3b(xr, bf, pm, p2t, cnt, W_v, bv2, g2, be2):
    return pl.pallas_call(
        _s3b_body,
        grid=(NB,),
        in_specs=[
            pl.BlockSpec((R, D), lambda i: (i, 0)),
            pl.BlockSpec((R, 1), lambda i: (i, 0)),
            pl.BlockSpec((R, K), lambda i: (i, 0)),
            pl.BlockSpec((D, B * K), lambda i: (0, 0)),
            pl.BlockSpec((1, B * K), lambda i: (0, 0)),
            pl.BlockSpec((D, D), lambda i: (0, 0)),
            pl.BlockSpec((1, D), lambda i: (0, 0)),
            pl.BlockSpec((1, D), lambda i: (0, 0)),
            pl.BlockSpec((1, D), lambda i: (0, 0)),
        ],
        out_specs=pl.BlockSpec((R, D), lambda i: (i, 0)),
        out_shape=jax.ShapeDtypeStruct((N, D), jnp.float32),
    )(xr, bf, pm, p2t, cnt, W_v, bv2, g2, be2)


def kernel(x, edge_index, edge_attr, pkt_mask, batch,
           W_e, b_e, eps, W1, b1, W2, b2, gamma, beta, W_v, b_v):
    src = edge_index[0]
    dst = edge_index[1]
    emb = _edge_emb(edge_attr, W_e, b_e.reshape(1, D))
    zeros = jnp.zeros((NP, D), jnp.float32)
    aggp = _sc_agg(x, src, dst, emb, zeros)
    a0 = aggp[:N]
    a1 = aggp[NP:NP + N]
    bf = batch.astype(jnp.float32).reshape(N, 1)
    eps2 = eps.reshape(1, 1)
    xr, p2t, cnt = _stage3a(x, a0, a1, bf, pkt_mask, eps2,
                            W1, b1.reshape(1, D), W2, b2.reshape(1, D))
    return _stage3b(xr, bf, pkt_mask, p2t, cnt,
                    W_v, b_v.reshape(1, D), gamma.reshape(1, D),
                    beta.reshape(1, D))


# speedup vs baseline: 4.1799x; 1.6085x over previous
import functools

import jax
import jax.numpy as jnp
from jax import lax
from jax.experimental import pallas as pl
from jax.experimental.pallas import tpu as pltpu
from jax.experimental.pallas import tpu_sc as plsc

N = 10000
E = 320000
D = 128
DE = 16
K = 8
B = 16

NC = 2
NS = 16
L = 16
NW = NC * NS
EPW = E // NW
C = 80
NCH = EPW // C
NP = 10240
RPS = NP // NS

R = 1000
NB = N // R


def _emb_body(a_ref, w_ref, b_ref, o_ref):
    o_ref[...] = (
        jnp.dot(a_ref[...], w_ref[...], preferred_element_type=jnp.float32)
        + b_ref[...]
    )


def _edge_emb(edge_attr, W_e, b_e2):
    BE = 8000
    return pl.pallas_call(
        _emb_body,
        grid=(E // BE,),
        in_specs=[
            pl.BlockSpec((BE, DE), lambda i: (i, 0)),
            pl.BlockSpec((DE, D), lambda i: (0, 0)),
            pl.BlockSpec((1, D), lambda i: (0, 0)),
        ],
        out_specs=pl.BlockSpec((BE, D), lambda i: (i, 0)),
        out_shape=jax.ShapeDtypeStruct((E, D), jnp.float32),
    )(edge_attr, W_e, b_e2)


NBUF = 2
GOUT = (NCH - 1) // NBUF


def _sc_body(x_hbm, src_hbm, dst_hbm, emb_hbm, zero_hbm, out_hbm, *scr):
    srcs = scr[0:NBUF]
    dsts = scr[NBUF:2 * NBUF]
    rows = scr[2 * NBUF:3 * NBUF]
    embs = scr[3 * NBUF:4 * NBUF]
    agg_sh = scr[4 * NBUF]
    sio = scr[4 * NBUF + 1:4 * NBUF + 1 + NBUF]
    sd = scr[4 * NBUF + 1 + NBUF:4 * NBUF + 1 + 2 * NBUF]
    sg = scr[4 * NBUF + 1 + 2 * NBUF:4 * NBUF + 1 + 3 * NBUF]
    ss = scr[4 * NBUF + 1 + 3 * NBUF:4 * NBUF + 1 + 4 * NBUF]

    cid = lax.axis_index("c")
    sid = lax.axis_index("s")
    wid = cid * NS + sid
    row0 = sid * RPS

    pltpu.sync_copy(zero_hbm.at[pl.ds(row0, RPS)],
                    agg_sh.at[pl.ds(row0, RPS)])
    plsc.subcore_barrier()

    e0 = wid * EPW

    def compute(b):
        def row(i, c2):
            for jj in range(D // L):
                sl = pl.ds(jj * L, L)
                rows[b][i, sl] = jnp.maximum(
                    rows[b][i, sl] + embs[b][i, sl], 0.0)
            return c2
        lax.fori_loop(0, C, row, 0)

    pltpu.sync_copy(src_hbm.at[pl.ds(e0, C)], srcs[0])
    pltpu.sync_copy(dst_hbm.at[pl.ds(e0, C)], dsts[0])
    pltpu.sync_copy(emb_hbm.at[pl.ds(e0, C)], embs[0])
    pltpu.async_copy(x_hbm.at[srcs[0]], rows[0], sg[0]).wait()
    compute(0)
    pltpu.async_copy(rows[0], agg_sh.at[dsts[0]], ss[0], add=True).wait()

    eb = e0 + C

    for b in range(NBUF):
        pltpu.async_copy(src_hbm.at[pl.ds(eb + b * C, C)], srcs[b], sio[b])
        pltpu.async_copy(emb_hbm.at[pl.ds(eb + b * C, C)], embs[b], sio[b])

    def outer(g, carry):
        for b in range(NBUF):
            t = g * NBUF + b
            off = eb + t * C

            @pl.when(g > 0)
            def _(b=b):
                pltpu.make_async_copy(rows[b], agg_sh.at[dsts[b]],
                                      ss[b]).wait()

            pltpu.async_copy(dst_hbm.at[pl.ds(off, C)], dsts[b], sd[b])
            pltpu.make_async_copy(src_hbm.at[pl.ds(off, C)], srcs[b],
                                  sio[b]).wait()
            pltpu.make_async_copy(emb_hbm.at[pl.ds(off, C)], embs[b],
                                  sio[b]).wait()
            pltpu.async_copy(x_hbm.at[srcs[b]], rows[b], sg[b])

        for b in range(NBUF):
            t = g * NBUF + b
            off_t = eb + t * C
            pltpu.make_async_copy(x_hbm.at[srcs[b]], rows[b], sg[b]).wait()
            compute(b)
            pltpu.make_async_copy(dst_hbm.at[pl.ds(off_t, C)], dsts[b],
                                  sd[b]).wait()
            pltpu.async_copy(rows[b], agg_sh.at[dsts[b]], ss[b], add=True)

            @pl.when(g < GOUT - 1)
            def _(b=b, t=t):
                off2 = eb + (t + NBUF) * C
                pltpu.async_copy(src_hbm.at[pl.ds(off2, C)], srcs[b], sio[b])
                pltpu.async_copy(emb_hbm.at[pl.ds(off2, C)], embs[b], sio[b])
        return carry

    lax.fori_loop(0, GOUT, outer, 0)
    for b in range(NBUF):
        pltpu.make_async_copy(rows[b], agg_sh.at[dsts[b]], ss[b]).wait()
    plsc.subcore_barrier()
    pltpu.sync_copy(agg_sh.at[pl.ds(row0, RPS)],
                    out_hbm.at[pl.ds(cid * NP + row0, RPS)])


_sc_agg = functools.partial(
    pl.kernel,
    out_type=jax.ShapeDtypeStruct((NC * NP, D), jnp.float32),
    mesh=plsc.VectorSubcoreMesh(
        core_axis_name="c", subcore_axis_name="s",
        num_cores=NC, num_subcores=NS),
    scratch_types=(
        [pltpu.VMEM((C,), jnp.int32) for _ in range(NBUF)]
        + [pltpu.VMEM((C,), jnp.int32) for _ in range(NBUF)]
        + [pltpu.VMEM((C, D), jnp.float32) for _ in range(NBUF)]
        + [pltpu.VMEM((C, D), jnp.float32) for _ in range(NBUF)]
        + [pltpu.VMEM_SHARED((NP, D), jnp.float32)]
        + [pltpu.SemaphoreType.DMA for _ in range(4 * NBUF)]
    ),
)(_sc_body)


def _mask_matrix(bf, pm):
    c = lax.broadcasted_iota(jnp.int32, (R, B * K), 1)
    onehot = bf == (c // K).astype(jnp.float32)
    pmt = jnp.concatenate([pm] * B, axis=1)
    return jnp.where(onehot, pmt, 0.0)


def _s3a_body(x_ref, a0_ref, a1_ref, bf_ref, pm_ref, eps_ref,
              w1_ref, b1_ref, w2_ref, b2_ref,
              xr_ref, p2t_ref, cnt_ref):
    i = pl.program_id(0)
    h = x_ref[...] * (1.0 + eps_ref[0, 0]) + a0_ref[...] + a1_ref[...]
    t = jnp.maximum(
        jnp.dot(h, w1_ref[...], preferred_element_type=jnp.float32)
        + b1_ref[...], 0.0)
    xr = (jnp.dot(t, w2_ref[...], preferred_element_type=jnp.float32)
          + b2_ref[...])
    xr_ref[...] = xr

    M = _mask_matrix(bf_ref[...], pm_ref[...])

    @pl.when(i == 0)
    def _():
        p2t_ref[...] = jnp.zeros_like(p2t_ref)
        cnt_ref[...] = jnp.zeros_like(cnt_ref)

    p2t_ref[...] += lax.dot_general(
        xr, M, (((0,), (0,)), ((), ())), preferred_element_type=jnp.float32)
    cnt_ref[...] += jnp.sum(M, axis=0, keepdims=True)


def _s3b_body(xr_ref, bf_ref, pm_ref, p2t_ref, cnt_ref,
              wv_ref, bv_ref, g_ref, be_ref, o_ref):
    recip = 1.0 / (cnt_ref[...] + 1e-9)
    pmean_t = p2t_ref[...] * recip
    pe = lax.dot_general(
        pmean_t, wv_ref[...], (((0,), (0,)), ((), ())),
        preferred_element_type=jnp.float32) + bv_ref[...]
    M = _mask_matrix(bf_ref[...], pm_ref[...])
    v = xr_ref[...] + jnp.dot(M, pe, preferred_element_type=jnp.float32)
    mu = jnp.mean(v, axis=1, keepdims=True)
    d = v - mu
    var = jnp.mean(d * d, axis=1, keepdims=True)
    o = d * lax.rsqrt(var + 1e-5) * g_ref[...] + be_ref[...]
    o_ref[...] = jnp.maximum(o, 0.0)


def _stage3a(x, a0, a1, bf, pm, eps2, W1, b12, W2, b22):
    return pl.pallas_call(
        _s3a_body,
        grid=(NB,),
        in_specs=[
            pl.BlockSpec((R, D), lambda i: (i, 0)),
            pl.BlockSpec((R, D), lambda i: (i, 0)),
            pl.BlockSpec((R, D), lambda i: (i, 0)),
            pl.BlockSpec((R, 1), lambda i: (i, 0)),
            pl.BlockSpec((R, K), lambda i: (i, 0)),
            pl.BlockSpec((1, 1), lambda i: (0, 0)),
            pl.BlockSpec((D, D), lambda i: (0, 0)),
            pl.BlockSpec((1, D), lambda i: (0, 0)),
            pl.BlockSpec((D, D), lambda i: (0, 0)),
            pl.BlockSpec((1, D), lambda i: (0, 0)),
        ],
        out_specs=[
            pl.BlockSpec((R, D), lambda i: (i, 0)),
            pl.BlockSpec((D, B * K), lambda i: (0, 0)),
            pl.BlockSpec((1, B * K), lambda i: (0, 0)),
        ],
        out_shape=[
            jax.ShapeDtypeStruct((N, D), jnp.float32),
            jax.ShapeDtypeStruct((D, B * K), jnp.float32),
            jax.ShapeDtypeStruct((1, B * K), jnp.float32),
        ],
    )(x, a0, a1, bf, pm, eps2, W1, b12, W2, b22)


def _stage3b(xr, bf, pm, p2t, cnt, W_v, bv2, g2, be2):
    return pl.pallas_call(
        _s3b_body,
        grid=(NB,),
        in_specs=[
            pl.BlockSpec((R, D), lambda i: (i, 0)),
            pl.BlockSpec((R, 1), lambda i: (i, 0)),
            pl.BlockSpec((R, K), lambda i: (i, 0)),
            pl.BlockSpec((D, B * K), lambda i: (0, 0)),
            pl.BlockSpec((1, B * K), lambda i: (0, 0)),
            pl.BlockSpec((D, D), lambda i: (0, 0)),
            pl.BlockSpec((1, D), lambda i: (0, 0)),
            pl.BlockSpec((1, D), lambda i: (0, 0)),
            pl.BlockSpec((1, D), lambda i: (0, 0)),
        ],
        out_specs=pl.BlockSpec((R, D), lambda i: (i, 0)),
        out_shape=jax.ShapeDtypeStruct((N, D), jnp.float32),
    )(xr, bf, pm, p2t, cnt, W_v, bv2, g2, be2)


def kernel(x, edge_index, edge_attr, pkt_mask, batch,
           W_e, b_e, eps, W1, b1, W2, b2, gamma, beta, W_v, b_v):
    src = edge_index[0]
    dst = edge_index[1]
    emb = _edge_emb(edge_attr, W_e, b_e.reshape(1, D))
    zeros = jnp.zeros((NP, D), jnp.float32)
    aggp = _sc_agg(x, src, dst, emb, zeros)
    a0 = aggp[:N]
    a1 = aggp[NP:NP + N]
    bf = batch.astype(jnp.float32).reshape(N, 1)
    eps2 = eps.reshape(1, 1)
    xr, p2t, cnt = _stage3a(x, a0, a1, bf, pkt_mask, eps2,
                            W1, b1.reshape(1, D), W2, b2.reshape(1, D))
    return _stage3b(xr, bf, pkt_mask, p2t, cnt,
                    W_v, b_v.reshape(1, D), gamma.reshape(1, D),
                    beta.reshape(1, D))


# speedup vs baseline: 4.3848x; 1.0490x over previous
import functools

import jax
import jax.numpy as jnp
from jax import lax
from jax.experimental import pallas as pl
from jax.experimental.pallas import tpu as pltpu
from jax.experimental.pallas import tpu_sc as plsc

N = 10000
E = 320000
D = 128
DE = 16
K = 8
B = 16

NC = 2
NS = 16
L = 16
NW = NC * NS
EPW = E // NW
C = 80
NCH = EPW // C
NP = 10240
RPS = NP // NS

R = 1000
NB = N // R


def _emb_body(a_ref, w_ref, b_ref, o_ref):
    o_ref[...] = (
        jnp.dot(a_ref[...], w_ref[...], preferred_element_type=jnp.float32)
        + b_ref[...]
    )


def _edge_emb(edge_attr, W_e, b_e2, ne, be):
    return pl.pallas_call(
        _emb_body,
        grid=(ne // be,),
        in_specs=[
            pl.BlockSpec((be, DE), lambda i: (i, 0)),
            pl.BlockSpec((DE, D), lambda i: (0, 0)),
            pl.BlockSpec((1, D), lambda i: (0, 0)),
        ],
        out_specs=pl.BlockSpec((be, D), lambda i: (i, 0)),
        out_shape=jax.ShapeDtypeStruct((ne, D), jnp.float32),
    )(edge_attr, W_e, b_e2)


NBUF = 2


def _make_sc_agg(epw):
    nch = epw // C
    gout = nch // NBUF
    pro = nch - gout * NBUF

    def _sc_body(x_hbm, src_hbm, dst_hbm, emb_hbm, init_hbm, out_hbm, *scr):
        srcs = scr[0:NBUF]
        dsts = scr[NBUF:2 * NBUF]
        rows = scr[2 * NBUF:3 * NBUF]
        embs = scr[3 * NBUF:4 * NBUF]
        agg_sh = scr[4 * NBUF]
        sio = scr[4 * NBUF + 1:4 * NBUF + 1 + NBUF]
        sd = scr[4 * NBUF + 1 + NBUF:4 * NBUF + 1 + 2 * NBUF]
        sg = scr[4 * NBUF + 1 + 2 * NBUF:4 * NBUF + 1 + 3 * NBUF]
        ss = scr[4 * NBUF + 1 + 3 * NBUF:4 * NBUF + 1 + 4 * NBUF]

        cid = lax.axis_index("c")
        sid = lax.axis_index("s")
        wid = cid * NS + sid
        row0 = sid * RPS

        pltpu.sync_copy(init_hbm.at[pl.ds(cid * NP + row0, RPS)],
                        agg_sh.at[pl.ds(row0, RPS)])
        plsc.subcore_barrier()

        e0 = wid * epw

        def compute(b):
            def row(i, c2):
                for jj in range(D // L):
                    sl = pl.ds(jj * L, L)
                    rows[b][i, sl] = jnp.maximum(
                        rows[b][i, sl] + embs[b][i, sl], 0.0)
                return c2
            lax.fori_loop(0, C, row, 0)

        for p in range(pro):
            po = e0 + p * C
            pltpu.sync_copy(src_hbm.at[pl.ds(po, C)], srcs[0])
            pltpu.sync_copy(dst_hbm.at[pl.ds(po, C)], dsts[0])
            pltpu.sync_copy(emb_hbm.at[pl.ds(po, C)], embs[0])
            pltpu.async_copy(x_hbm.at[srcs[0]], rows[0], sg[0]).wait()
            compute(0)
            pltpu.async_copy(rows[0], agg_sh.at[dsts[0]], ss[0],
                             add=True).wait()

        eb = e0 + pro * C

        for b in range(NBUF):
            pltpu.async_copy(src_hbm.at[pl.ds(eb + b * C, C)], srcs[b],
                             sio[b])
            pltpu.async_copy(emb_hbm.at[pl.ds(eb + b * C, C)], embs[b],
                             sio[b])

        def outer(g, carry):
            for b in range(NBUF):
                t = g * NBUF + b
                off = eb + t * C

                @pl.when(g > 0)
                def _(b=b):
                    pltpu.make_async_copy(rows[b], agg_sh.at[dsts[b]],
                                          ss[b]).wait()

                pltpu.async_copy(dst_hbm.at[pl.ds(off, C)], dsts[b], sd[b])
                pltpu.make_async_copy(src_hbm.at[pl.ds(off, C)], srcs[b],
                                      sio[b]).wait()
                pltpu.make_async_copy(emb_hbm.at[pl.ds(off, C)], embs[b],
                                      sio[b]).wait()
                pltpu.async_copy(x_hbm.at[srcs[b]], rows[b], sg[b])

            for b in range(NBUF):
                t = g * NBUF + b
                off_t = eb + t * C
                pltpu.make_async_copy(x_hbm.at[srcs[b]], rows[b],
                                      sg[b]).wait()
                compute(b)
                pltpu.make_async_copy(dst_hbm.at[pl.ds(off_t, C)], dsts[b],
                                      sd[b]).wait()
                pltpu.async_copy(rows[b], agg_sh.at[dsts[b]], ss[b],
                                 add=True)

                @pl.when(g < gout - 1)
                def _(b=b, t=t):
                    off2 = eb + (t + NBUF) * C
                    pltpu.async_copy(src_hbm.at[pl.ds(off2, C)], srcs[b],
                                     sio[b])
                    pltpu.async_copy(emb_hbm.at[pl.ds(off2, C)], embs[b],
                                     sio[b])
            return carry

        lax.fori_loop(0, gout, outer, 0)
        for b in range(NBUF):
            pltpu.make_async_copy(rows[b], agg_sh.at[dsts[b]], ss[b]).wait()
        plsc.subcore_barrier()
        pltpu.sync_copy(agg_sh.at[pl.ds(row0, RPS)],
                        out_hbm.at[pl.ds(cid * NP + row0, RPS)])

    return functools.partial(
        pl.kernel,
        out_type=jax.ShapeDtypeStruct((NC * NP, D), jnp.float32),
        mesh=plsc.VectorSubcoreMesh(
            core_axis_name="c", subcore_axis_name="s",
            num_cores=NC, num_subcores=NS),
        scratch_types=(
            [pltpu.VMEM((C,), jnp.int32) for _ in range(NBUF)]
            + [pltpu.VMEM((C,), jnp.int32) for _ in range(NBUF)]
            + [pltpu.VMEM((C, D), jnp.float32) for _ in range(NBUF)]
            + [pltpu.VMEM((C, D), jnp.float32) for _ in range(NBUF)]
            + [pltpu.VMEM_SHARED((NP, D), jnp.float32)]
            + [pltpu.SemaphoreType.DMA for _ in range(4 * NBUF)]
        ),
    )(_sc_body)


EPW_A = 4960
EPW_B = 5040
E_A = EPW_A * NW
E_B = EPW_B * NW
_sc_agg_a = _make_sc_agg(EPW_A)
_sc_agg_b = _make_sc_agg(EPW_B)


def _mask_matrix(bf, pm):
    c = lax.broadcasted_iota(jnp.int32, (R, B * K), 1)
    onehot = bf == (c // K).astype(jnp.float32)
    pmt = jnp.concatenate([pm] * B, axis=1)
    return jnp.where(onehot, pmt, 0.0)


def _s3a_body(x_ref, a0_ref, a1_ref, bf_ref, pm_ref, eps_ref,
              w1_ref, b1_ref, w2_ref, b2_ref,
              xr_ref, p2t_ref, cnt_ref):
    i = pl.program_id(0)
    h = x_ref[...] * (1.0 + eps_ref[0, 0]) + a0_ref[...] + a1_ref[...]
    t = jnp.maximum(
        jnp.dot(h, w1_ref[...], preferred_element_type=jnp.float32)
        + b1_ref[...], 0.0)
    xr = (jnp.dot(t, w2_ref[...], preferred_element_type=jnp.float32)
          + b2_ref[...])
    xr_ref[...] = xr

    M = _mask_matrix(bf_ref[...], pm_ref[...])

    @pl.when(i == 0)
    def _():
        p2t_ref[...] = jnp.zeros_like(p2t_ref)
        cnt_ref[...] = jnp.zeros_like(cnt_ref)

    p2t_ref[...] += lax.dot_general(
        xr, M, (((0,), (0,)), ((), ())), preferred_element_type=jnp.float32)
    cnt_ref[...] += jnp.sum(M, axis=0, keepdims=True)


def _s3b_body(xr_ref, bf_ref, pm_ref, p2t_ref, cnt_ref,
              wv_ref, bv_ref, g_ref, be_ref, o_ref):
    recip = 1.0 / (cnt_ref[...] + 1e-9)
    pmean_t = p2t_ref[...] * recip
    pe = lax.dot_general(
        pmean_t, wv_ref[...], (((0,), (0,)), ((), ())),
        preferred_element_type=jnp.float32) + bv_ref[...]
    M = _mask_matrix(bf_ref[...], pm_ref[...])
    v = xr_ref[...] + jnp.dot(M, pe, preferred_element_type=jnp.float32)
    mu = jnp.mean(v, axis=1, keepdims=True)
    d = v - mu
    var = jnp.mean(d * d, axis=1, keepdims=True)
    o = d * lax.rsqrt(var + 1e-5) * g_ref[...] + be_ref[...]
    o_ref[...] = jnp.maximum(o, 0.0)


def _stage3a(x, a0, a1, bf, pm, eps2, W1, b12, W2, b22):
    return pl.pallas_call(
        _s3a_body,
        grid=(NB,),
        in_specs=[
            pl.BlockSpec((R, D), lambda i: (i, 0)),
            pl.BlockSpec((R, D), lambda i: (i, 0)),
            pl.BlockSpec((R, D), lambda i: (i, 0)),
            pl.BlockSpec((R, 1), lambda i: (i, 0)),
            pl.BlockSpec((R, K), lambda i: (i, 0)),
            pl.BlockSpec((1, 1), lambda i: (0, 0)),
            pl.BlockSpec((D, D), lambda i: (0, 0)),
            pl.BlockSpec((1, D), lambda i: (0, 0)),
            pl.BlockSpec((D, D), lambda i: (0, 0)),
            pl.BlockSpec((1, D), lambda i: (0, 0)),
        ],
        out_specs=[
            pl.BlockSpec((R, D), lambda i: (i, 0)),
            pl.BlockSpec((D, B * K), lambda i: (0, 0)),
            pl.BlockSpec((1, B * K), lambda i: (0, 0)),
        ],
        out_shape=[
            jax.ShapeDtypeStruct((N, D), jnp.float32),
            jax.ShapeDtypeStruct((D, B * K), jnp.float32),
            jax.ShapeDtypeStruct((1, B * K), jnp.float32),
        ],
    )(x, a0, a1, bf, pm, eps2, W1, b12, W2, b22)


def _stage3b(xr, bf, pm, p2t, cnt, W_v, bv2, g2, be2):
    return pl.pallas_call(
        _s3b_body,
        grid=(NB,),
        in_specs=[
            pl.BlockSpec((R, D), lambda i: (i, 0)),
            pl.BlockSpec((R, 1), lambda i: (i, 0)),
            pl.BlockSpec((R, K), lambda i: (i, 0)),
            pl.BlockSpec((D, B * K), lambda i: (0, 0)),
            pl.BlockSpec((1, B * K), lambda i: (0, 0)),
            pl.BlockSpec((D, D), lambda i: (0, 0)),
            pl.BlockSpec((1, D), lambda i: (0, 0)),
            pl.BlockSpec((1, D), lambda i: (0, 0)),
            pl.BlockSpec((1, D), lambda i: (0, 0)),
        ],
        out_specs=pl.BlockSpec((R, D), lambda i: (i, 0)),
        out_shape=jax.ShapeDtypeStruct((N, D), jnp.float32),
    )(xr, bf, pm, p2t, cnt, W_v, bv2, g2, be2)


def kernel(x, edge_index, edge_attr, pkt_mask, batch,
           W_e, b_e, eps, W1, b1, W2, b2, gamma, beta, W_v, b_v):
    src = edge_index[0]
    dst = edge_index[1]
    b_e2 = b_e.reshape(1, D)
    emb1 = _edge_emb(edge_attr[:E_A], W_e, b_e2, E_A, EPW_A)
    emb2 = _edge_emb(edge_attr[E_A:], W_e, b_e2, E_B, EPW_B)
    zeros = jnp.zeros((NC * NP, D), jnp.float32)
    agg1 = _sc_agg_a(x, src[:E_A], dst[:E_A], emb1, zeros)
    aggp = _sc_agg_b(x, src[E_A:], dst[E_A:], emb2, agg1)
    a0 = aggp[:N]
    a1 = aggp[NP:NP + N]
    bf = batch.astype(jnp.float32).reshape(N, 1)
    eps2 = eps.reshape(1, 1)
    xr, p2t, cnt = _stage3a(x, a0, a1, bf, pkt_mask, eps2,
                            W1, b1.reshape(1, D), W2, b2.reshape(1, D))
    return _stage3b(xr, bf, pkt_mask, p2t, cnt,
                    W_v, b_v.reshape(1, D), gamma.reshape(1, D),
                    beta.reshape(1, D))


# speedup vs baseline: 4.7781x; 1.0897x over previous
import functools

import jax
import jax.numpy as jnp
from jax import lax
from jax.experimental import pallas as pl
from jax.experimental.pallas import tpu as pltpu
from jax.experimental.pallas import tpu_sc as plsc

N = 10000
E = 320000
D = 128
DE = 16
K = 8
B = 16

NC = 2
NS = 16
L = 16
NW = NC * NS
EPW = E // NW
C = 80
NCH = EPW // C
NP = 10240
RPS = NP // NS

R = 1000
NB = N // R


def _emb_body(a_ref, w_ref, b_ref, o_ref):
    o_ref[...] = (
        jnp.dot(a_ref[...], w_ref[...], preferred_element_type=jnp.float32)
        + b_ref[...]
    )


def _edge_emb(edge_attr, W_e, b_e2, ne, be):
    return pl.pallas_call(
        _emb_body,
        grid=(ne // be,),
        in_specs=[
            pl.BlockSpec((be, DE), lambda i: (i, 0)),
            pl.BlockSpec((DE, D), lambda i: (0, 0)),
            pl.BlockSpec((1, D), lambda i: (0, 0)),
        ],
        out_specs=pl.BlockSpec((be, D), lambda i: (i, 0)),
        out_shape=jax.ShapeDtypeStruct((ne, D), jnp.float32),
    )(edge_attr, W_e, b_e2)


NBUF = 2


def _make_sc_agg(epw, seed_from_init):
    nch = epw // C
    gout = nch // NBUF
    pro = nch - gout * NBUF

    def _sc_body(x_hbm, src_hbm, dst_hbm, emb_hbm, init_hbm, out_hbm, *scr):
        srcs = scr[0:NBUF]
        dsts = scr[NBUF:2 * NBUF]
        rows = scr[2 * NBUF:3 * NBUF]
        embs = scr[3 * NBUF:4 * NBUF]
        agg_sh = scr[4 * NBUF]
        sio = scr[4 * NBUF + 1:4 * NBUF + 1 + NBUF]
        sd = scr[4 * NBUF + 1 + NBUF:4 * NBUF + 1 + 2 * NBUF]
        sg = scr[4 * NBUF + 1 + 2 * NBUF:4 * NBUF + 1 + 3 * NBUF]
        ss = scr[4 * NBUF + 1 + 3 * NBUF:4 * NBUF + 1 + 4 * NBUF]

        cid = lax.axis_index("c")
        sid = lax.axis_index("s")
        wid = cid * NS + sid
        row0 = sid * RPS

        if seed_from_init:
            pltpu.sync_copy(init_hbm.at[pl.ds(cid * NP + row0, RPS)],
                            agg_sh.at[pl.ds(row0, RPS)])
        else:
            def zrow(i, c2):
                for jj in range(D // L):
                    rows[0][i, pl.ds(jj * L, L)] = jnp.zeros((L,),
                                                             jnp.float32)
                return c2
            lax.fori_loop(0, C, zrow, 0)
            for q in range(RPS // C):
                pltpu.sync_copy(rows[0],
                                agg_sh.at[pl.ds(row0 + q * C, C)])
        plsc.subcore_barrier()

        e0 = wid * epw

        def compute(b):
            def row(j, c2):
                for u in range(2):
                    for jj in range(D // L):
                        sl = pl.ds(jj * L, L)
                        rows[b][2 * j + u, sl] = jnp.maximum(
                            rows[b][2 * j + u, sl]
                            + embs[b][2 * j + u, sl], 0.0)
                return c2
            lax.fori_loop(0, C // 2, row, 0)

        for p in range(pro):
            po = e0 + p * C
            pltpu.sync_copy(src_hbm.at[pl.ds(po, C)], srcs[0])
            pltpu.sync_copy(dst_hbm.at[pl.ds(po, C)], dsts[0])
            pltpu.sync_copy(emb_hbm.at[pl.ds(po, C)], embs[0])
            pltpu.async_copy(x_hbm.at[srcs[0]], rows[0], sg[0]).wait()
            compute(0)
            pltpu.async_copy(rows[0], agg_sh.at[dsts[0]], ss[0],
                             add=True).wait()

        eb = e0 + pro * C

        for b in range(NBUF):
            pltpu.async_copy(src_hbm.at[pl.ds(eb + b * C, C)], srcs[b],
                             sio[b])
            pltpu.async_copy(emb_hbm.at[pl.ds(eb + b * C, C)], embs[b],
                             sio[b])

        def outer(g, carry):
            for b in range(NBUF):
                t = g * NBUF + b
                off = eb + t * C

                @pl.when(g > 0)
                def _(b=b):
                    pltpu.make_async_copy(rows[b], agg_sh.at[dsts[b]],
                                          ss[b]).wait()

                pltpu.async_copy(dst_hbm.at[pl.ds(off, C)], dsts[b], sd[b])
                pltpu.make_async_copy(src_hbm.at[pl.ds(off, C)], srcs[b],
                                      sio[b]).wait()
                pltpu.make_async_copy(emb_hbm.at[pl.ds(off, C)], embs[b],
                                      sio[b]).wait()
                pltpu.async_copy(x_hbm.at[srcs[b]], rows[b], sg[b])

            for b in range(NBUF):
                t = g * NBUF + b
                off_t = eb + t * C
                pltpu.make_async_copy(x_hbm.at[srcs[b]], rows[b],
                                      sg[b]).wait()
                compute(b)
                pltpu.make_async_copy(dst_hbm.at[pl.ds(off_t, C)], dsts[b],
                                      sd[b]).wait()
                pltpu.async_copy(rows[b], agg_sh.at[dsts[b]], ss[b],
                                 add=True)

                @pl.when(g < gout - 1)
                def _(b=b, t=t):
                    off2 = eb + (t + NBUF) * C
                    pltpu.async_copy(src_hbm.at[pl.ds(off2, C)], srcs[b],
                                     sio[b])
                    pltpu.async_copy(emb_hbm.at[pl.ds(off2, C)], embs[b],
                                     sio[b])
            return carry

        lax.fori_loop(0, gout, outer, 0)
        for b in range(NBUF):
            pltpu.make_async_copy(rows[b], agg_sh.at[dsts[b]], ss[b]).wait()
        plsc.subcore_barrier()
        pltpu.sync_copy(agg_sh.at[pl.ds(row0, RPS)],
                        out_hbm.at[pl.ds(cid * NP + row0, RPS)])

    return functools.partial(
        pl.kernel,
        out_type=jax.ShapeDtypeStruct((NC * NP, D), jnp.float32),
        mesh=plsc.VectorSubcoreMesh(
            core_axis_name="c", subcore_axis_name="s",
            num_cores=NC, num_subcores=NS),
        scratch_types=(
            [pltpu.VMEM((C,), jnp.int32) for _ in range(NBUF)]
            + [pltpu.VMEM((C,), jnp.int32) for _ in range(NBUF)]
            + [pltpu.VMEM((C, D), jnp.float32) for _ in range(NBUF)]
            + [pltpu.VMEM((C, D), jnp.float32) for _ in range(NBUF)]
            + [pltpu.VMEM_SHARED((NP, D), jnp.float32)]
            + [pltpu.SemaphoreType.DMA for _ in range(4 * NBUF)]
        ),
    )(_sc_body)


EPW_A = 4960
EPW_B = 5040
E_A = EPW_A * NW
E_B = EPW_B * NW
_sc_agg_a = _make_sc_agg(EPW_A, seed_from_init=False)
_sc_agg_b = _make_sc_agg(EPW_B, seed_from_init=True)


def _mask_matrix(bf, pmt):
    c = lax.broadcasted_iota(jnp.int32, (R, B * K), 1)
    onehot = bf == (c // K).astype(jnp.float32)
    return jnp.where(onehot, pmt, 0.0)


def _s3_body(x_ref, a0_ref, a1_ref, bf_ref, pm_ref, eps_ref,
             w1_ref, b1_ref, w2_ref, b2_ref, wv_ref, bv_ref,
             g_ref, be_ref, o_ref, xr_s, p2t_s, cnt_s):
    p = pl.program_id(0)
    i = pl.program_id(1)
    M = _mask_matrix(bf_ref[...], pm_ref[...])

    @pl.when(p == 0)
    def _():
        h = (x_ref[...] * (1.0 + eps_ref[0, 0])
             + a0_ref[...] + a1_ref[...])
        t = jnp.maximum(
            jnp.dot(h, w1_ref[...], preferred_element_type=jnp.float32)
            + b1_ref[...], 0.0)
        xr = (jnp.dot(t, w2_ref[...], preferred_element_type=jnp.float32)
              + b2_ref[...])
        xr_s[pl.ds(i * R, R), :] = xr

        @pl.when(i == 0)
        def _():
            p2t_s[...] = jnp.zeros_like(p2t_s)
            cnt_s[...] = jnp.zeros_like(cnt_s)

        p2t_s[...] += lax.dot_general(
            xr, M, (((0,), (0,)), ((), ())),
            preferred_element_type=jnp.float32)
        cnt_s[...] += jnp.sum(M, axis=0, keepdims=True)

    @pl.when(p == 1)
    def _():
        recip = 1.0 / (cnt_s[...] + 1e-9)
        pmean_t = p2t_s[...] * recip
        pe = lax.dot_general(
            pmean_t, wv_ref[...], (((0,), (0,)), ((), ())),
            preferred_element_type=jnp.float32) + bv_ref[...]
        xr = xr_s[pl.ds(i * R, R), :]
        v = xr + jnp.dot(M, pe, preferred_element_type=jnp.float32)
        mu = jnp.mean(v, axis=1, keepdims=True)
        d = v - mu
        var = jnp.mean(d * d, axis=1, keepdims=True)
        o = d * lax.rsqrt(var + 1e-5) * g_ref[...] + be_ref[...]
        o_ref[...] = jnp.maximum(o, 0.0)


def _stage3(x, a0, a1, bf, pm, eps2, W1, b12, W2, b22, W_v, bv2, g2, be2):
    big = pl.BlockSpec((R, D), lambda p, i: (i * (1 - p), 0))
    row = pl.BlockSpec((R, D), lambda p, i: (i, 0))
    cst = lambda bs: pl.BlockSpec(bs, lambda p, i: (0, 0))
    return pl.pallas_call(
        _s3_body,
        grid=(2, NB),
        in_specs=[
            big, big, big,
            pl.BlockSpec((R, 1), lambda p, i: (i, 0)),
            pl.BlockSpec((R, B * K), lambda p, i: (i, 0)),
            cst((1, 1)), cst((D, D)), cst((1, D)), cst((D, D)),
            cst((1, D)), cst((D, D)), cst((1, D)), cst((1, D)),
            cst((1, D)),
        ],
        out_specs=pl.BlockSpec((R, D), lambda p, i: (i * p, 0)),
        out_shape=jax.ShapeDtypeStruct((N, D), jnp.float32),
        scratch_shapes=[
            pltpu.VMEM((N, D), jnp.float32),
            pltpu.VMEM((D, B * K), jnp.float32),
            pltpu.VMEM((1, B * K), jnp.float32),
        ],
    )(x, a0, a1, bf, pm, eps2, W1, b12, W2, b22, W_v, bv2, g2, be2)


def kernel(x, edge_index, edge_attr, pkt_mask, batch,
           W_e, b_e, eps, W1, b1, W2, b2, gamma, beta, W_v, b_v):
    src = edge_index[0]
    dst = edge_index[1]
    b_e2 = b_e.reshape(1, D)
    emb1 = _edge_emb(edge_attr[:E_A], W_e, b_e2, E_A, EPW_A)
    emb2 = _edge_emb(edge_attr[E_A:], W_e, b_e2, E_B, EPW_B)
    agg1 = _sc_agg_a(x, src[:E_A], dst[:E_A], emb1, x)
    aggp = _sc_agg_b(x, src[E_A:], dst[E_A:], emb2, agg1)
    a0 = aggp[:N]
    a1 = aggp[NP:NP + N]
    bf = batch.astype(jnp.float32).reshape(N, 1)
    pmt = jnp.tile(pkt_mask, (1, B))
    eps2 = eps.reshape(1, 1)
    return _stage3(x, a0, a1, bf, pmt, eps2,
                   W1, b1.reshape(1, D), W2, b2.reshape(1, D),
                   W_v, b_v.reshape(1, D), gamma.reshape(1, D),
                   beta.reshape(1, D))
